# Initial kernel scaffold; baseline (speedup 1.0000x reference)
import jax
import jax.numpy as jnp
from jax.experimental import pallas as pl


def kernel(x, adj, W1, b1, W2, b2):
    raise NotImplementedError("write your pallas kernel here")


# speedup vs baseline: 1.0131x; 1.0131x over previous
import functools

import jax
import jax.numpy as jnp
from jax.experimental import pallas as pl
from jax.experimental.pallas import tpu as pltpu

_BI = 400


def _pass_kernel(adj_ref, v_ref, w_ref, b_ref, out_ref, *, relu):
    acc = jnp.dot(adj_ref[...], v_ref[...], preferred_element_type=jnp.float32)
    r = jnp.dot(acc, w_ref[...], preferred_element_type=jnp.float32) + b_ref[...]
    if relu:
        r = jnp.maximum(r, 0.0)
    out_ref[...] = r


def _gcn_pass(adj, v, w, b2d, relu, bi=_BI):
    n, _ = adj.shape
    d = v.shape[1]
    return pl.pallas_call(
        functools.partial(_pass_kernel, relu=relu),
        grid=(n // bi,),
        in_specs=[
            pl.BlockSpec((bi, n), lambda i: (i, 0)),
            pl.BlockSpec((n, d), lambda i: (0, 0)),
            pl.BlockSpec(w.shape, lambda i: (0, 0)),
            pl.BlockSpec(b2d.shape, lambda i: (0, 0)),
        ],
        out_specs=pl.BlockSpec((bi, d), lambda i: (i, 0)),
        out_shape=jax.ShapeDtypeStruct((n, d), jnp.float32),
        compiler_params=pltpu.CompilerParams(
            dimension_semantics=("arbitrary",),
        ),
    )(adj, v, w, b2d)


def kernel(x, adj, W1, b1, W2, b2):
    h = _gcn_pass(adj, x, W1, b1.reshape(1, -1), relu=True)
    out = _gcn_pass(adj, h, W2, b2.reshape(1, -1), relu=False)
    return out


# speedup vs baseline: 1.0153x; 1.0022x over previous
import functools

import jax
import jax.numpy as jnp
from jax.experimental import pallas as pl
from jax.experimental.pallas import tpu as pltpu

_BI = 400


def _pass_kernel(adj_ref, v_ref, w_ref, b_ref, out_ref, *, relu):
    a = adj_ref[...].astype(jnp.bfloat16)
    v = v_ref[...].astype(jnp.bfloat16)
    acc = jnp.dot(a, v, preferred_element_type=jnp.float32)
    r = jnp.dot(acc, w_ref[...], preferred_element_type=jnp.float32) + b_ref[...]
    if relu:
        r = jnp.maximum(r, 0.0)
    out_ref[...] = r


def _gcn_pass(adj, v, w, b2d, relu, bi=_BI):
    n, _ = adj.shape
    d = v.shape[1]
    return pl.pallas_call(
        functools.partial(_pass_kernel, relu=relu),
        grid=(n // bi,),
        in_specs=[
            pl.BlockSpec((bi, n), lambda i: (i, 0)),
            pl.BlockSpec((n, d), lambda i: (0, 0)),
            pl.BlockSpec(w.shape, lambda i: (0, 0)),
            pl.BlockSpec(b2d.shape, lambda i: (0, 0)),
        ],
        out_specs=pl.BlockSpec((bi, d), lambda i: (i, 0)),
        out_shape=jax.ShapeDtypeStruct((n, d), jnp.float32),
        compiler_params=pltpu.CompilerParams(
            dimension_semantics=("arbitrary",),
        ),
    )(adj, v, w, b2d)


def kernel(x, adj, W1, b1, W2, b2):
    h = _gcn_pass(adj, x, W1, b1.reshape(1, -1), relu=True)
    out = _gcn_pass(adj, h, W2, b2.reshape(1, -1), relu=False)
    return out
